# baseline (device time: 261043 ns/iter reference)
import jax
import jax.numpy as jnp
from jax import lax
from jax.experimental import pallas as pl
from jax.experimental.pallas import tpu as pltpu

N = 32
M_PER = 512
D = 512
F = N // 2

PERM = [
    0, 3, 4, 7, 15, 12, 11, 8, 16, 19, 20, 23, 31, 28, 27, 24,
    25, 26, 29, 30, 22, 21, 18, 17, 9, 10, 13, 14, 6, 5, 2, 1,
]
assert sorted(PERM) == list(range(N))
INV = [0] * N
for _k, _l in enumerate(PERM):
    INV[_l] = _k


def _fused(kpos, perm, x16, w1, w2):
    def body(kpos_ref, perm_ref, x_ref, w1_ref, w2_ref, out_ref,
             xfull, pbuf, facc, bacc,
             agf_s, agf_r, agb_s, agb_r,
             rsf_s, rsf_r, rsb_s, rsb_r):
        i = lax.axis_index("i")
        k = kpos_ref[0]
        left = perm_ref[(k - 1) % N]
        right = perm_ref[(k + 1) % N]

        barrier = pltpu.get_barrier_semaphore()
        for nbr in (left, right):
            pl.semaphore_signal(
                barrier, inc=1, device_id=(nbr,),
                device_id_type=pl.DeviceIdType.MESH,
            )
        pl.semaphore_wait(barrier, 2)

        def chunk(ref, c):
            return ref.at[pl.ds(c * M_PER, M_PER), :]

        def compute(c):
            xc = xfull[pl.ds(c * M_PER, M_PER), :]
            hc = jnp.dot(xc, w1_ref[...], preferred_element_type=jnp.float32)
            hc = hc * jax.nn.sigmoid(hc)
            pc = jnp.dot(
                hc.astype(jnp.bfloat16), w2_ref[...],
                preferred_element_type=jnp.float32,
            )
            pbuf[pl.ds(c * M_PER, M_PER), :] = pc.astype(jnp.bfloat16)

        xfull[pl.ds(i * M_PER, M_PER), :] = x_ref[...]
        compute(i)
        for h in range(F):
            cf = perm_ref[(k - h) % N]
            rf = pltpu.make_async_remote_copy(
                src_ref=chunk(xfull, cf), dst_ref=chunk(xfull, cf),
                send_sem=agf_s.at[h], recv_sem=agf_r.at[h],
                device_id=(right,), device_id_type=pl.DeviceIdType.MESH,
            )
            rf.start()
            if h < F - 1:
                cb = perm_ref[(k + h) % N]
                rb = pltpu.make_async_remote_copy(
                    src_ref=chunk(xfull, cb), dst_ref=chunk(xfull, cb),
                    send_sem=agb_s.at[h], recv_sem=agb_r.at[h],
                    device_id=(left,), device_id_type=pl.DeviceIdType.MESH,
                )
                rb.start()
            if h >= 1:
                compute(perm_ref[(k - h) % N])
                compute(perm_ref[(k + h) % N])
            rf.wait()
            if h < F - 1:
                rb.wait()
        compute(perm_ref[(k - F) % N])

        facc[pl.ds(0, M_PER), :] = pbuf[
            pl.ds(perm_ref[(k + F - 1) % N] * M_PER, M_PER), :
        ]
        bacc[pl.ds(0, M_PER), :] = pbuf[
            pl.ds(perm_ref[(k - F) % N] * M_PER, M_PER), :
        ]
        for h in range(F):
            if h < F - 1:
                rf = pltpu.make_async_remote_copy(
                    src_ref=chunk(facc, h), dst_ref=chunk(facc, h + 1),
                    send_sem=rsf_s.at[h], recv_sem=rsf_r.at[h],
                    device_id=(right,), device_id_type=pl.DeviceIdType.MESH,
                )
                rf.start()
            rb = pltpu.make_async_remote_copy(
                src_ref=chunk(bacc, h), dst_ref=chunk(bacc, h + 1),
                send_sem=rsb_s.at[h], recv_sem=rsb_r.at[h],
                device_id=(left,), device_id_type=pl.DeviceIdType.MESH,
            )
            rb.start()
            if h < F - 1:
                rf.wait()
            rb.wait()
            if h < F - 2:
                cf_r = perm_ref[(k + F - 2 - h) % N]
                facc[pl.ds((h + 1) * M_PER, M_PER), :] = (
                    facc[pl.ds((h + 1) * M_PER, M_PER), :]
                    + pbuf[pl.ds(cf_r * M_PER, M_PER), :]
                )
            if h < F - 1:
                cb_r = perm_ref[(k - F + 1 + h) % N]
                bacc[pl.ds((h + 1) * M_PER, M_PER), :] = (
                    bacc[pl.ds((h + 1) * M_PER, M_PER), :]
                    + pbuf[pl.ds(cb_r * M_PER, M_PER), :]
                )
        out_ref[...] = (
            facc[pl.ds((F - 1) * M_PER, M_PER), :].astype(jnp.float32)
            + bacc[pl.ds(F * M_PER, M_PER), :].astype(jnp.float32)
            + pbuf[pl.ds(i * M_PER, M_PER), :].astype(jnp.float32)
        )

    return pl.pallas_call(
        body,
        out_shape=jax.ShapeDtypeStruct((M_PER, D), jnp.float32),
        in_specs=[
            pl.BlockSpec(memory_space=pltpu.SMEM),
            pl.BlockSpec(memory_space=pltpu.SMEM),
            pl.BlockSpec(memory_space=pltpu.VMEM),
            pl.BlockSpec(memory_space=pltpu.VMEM),
            pl.BlockSpec(memory_space=pltpu.VMEM),
        ],
        out_specs=pl.BlockSpec(memory_space=pltpu.VMEM),
        scratch_shapes=[
            pltpu.VMEM((N * M_PER, D), jnp.bfloat16),
            pltpu.VMEM((N * M_PER, D), jnp.bfloat16),
            pltpu.VMEM((F * M_PER, D), jnp.bfloat16),
            pltpu.VMEM(((F + 1) * M_PER, D), jnp.bfloat16),
            pltpu.SemaphoreType.DMA((F,)),
            pltpu.SemaphoreType.DMA((F,)),
            pltpu.SemaphoreType.DMA((F - 1,)),
            pltpu.SemaphoreType.DMA((F - 1,)),
            pltpu.SemaphoreType.DMA((F - 1,)),
            pltpu.SemaphoreType.DMA((F - 1,)),
            pltpu.SemaphoreType.DMA((F,)),
            pltpu.SemaphoreType.DMA((F,)),
        ],
        compiler_params=pltpu.CompilerParams(
            collective_id=0,
            vmem_limit_bytes=60 * 1024 * 1024,
        ),
    )(kpos, perm, x16, w1, w2)


def kernel(x, W1, W2):
    i = lax.axis_index("i")
    kpos = jnp.take(jnp.array(INV, dtype=jnp.int32), i).reshape(1)
    perm = jnp.array(PERM, dtype=jnp.int32)
    return _fused(
        kpos,
        perm,
        x.astype(jnp.bfloat16),
        W1.astype(jnp.bfloat16),
        W2.astype(jnp.bfloat16),
    )


# device time: 218801 ns/iter; 1.1931x vs baseline; 1.1931x over previous
import jax
import jax.numpy as jnp
from jax import lax
from jax.experimental import pallas as pl
from jax.experimental.pallas import tpu as pltpu

N = 32
M_PER = 512
D = 512
F = N // 2
SUB = 2
MS = M_PER // SUB

PERM = [
    0, 3, 4, 7, 15, 12, 11, 8, 16, 19, 20, 23, 31, 28, 27, 24,
    25, 26, 29, 30, 22, 21, 18, 17, 9, 10, 13, 14, 6, 5, 2, 1,
]
assert sorted(PERM) == list(range(N))
INV = [0] * N
for _k, _l in enumerate(PERM):
    INV[_l] = _k


def _fused(kpos, perm, x16, w1, w2):
    def body(kpos_ref, perm_ref, x_ref, w1_ref, w2_ref, out_ref,
             xfull, pbuf, facc, bacc,
             agf_s, agf_r, agb_s, agb_r,
             rsf_s, rsf_r, rsb_s, rsb_r):
        i = lax.axis_index("i")
        k = kpos_ref[0]
        left = perm_ref[(k - 1) % N]
        right = perm_ref[(k + 1) % N]

        barrier = pltpu.get_barrier_semaphore()
        for nbr in (left, right):
            pl.semaphore_signal(
                barrier, inc=1, device_id=(nbr,),
                device_id_type=pl.DeviceIdType.MESH,
            )
        pl.semaphore_wait(barrier, 2)

        def sub(ref, s, q):
            return ref.at[pl.ds(s * M_PER + q * MS, MS), :]

        def compute(c):
            xc = xfull[pl.ds(c * M_PER, M_PER), :]
            hc = jnp.dot(xc, w1_ref[...], preferred_element_type=jnp.float32)
            hc = hc * jax.nn.sigmoid(hc)
            pc = jnp.dot(
                hc.astype(jnp.bfloat16), w2_ref[...],
                preferred_element_type=jnp.float32,
            )
            pbuf[pl.ds(c * M_PER, M_PER), :] = pc.astype(jnp.bfloat16)

        def mk_agf(h, q):
            cf = perm_ref[(k - h) % N]
            return pltpu.make_async_remote_copy(
                src_ref=sub(xfull, cf, q), dst_ref=sub(xfull, cf, q),
                send_sem=agf_s.at[h, q], recv_sem=agf_r.at[h, q],
                device_id=(right,), device_id_type=pl.DeviceIdType.MESH,
            )

        def mk_agb(h, q):
            cb = perm_ref[(k + h) % N]
            return pltpu.make_async_remote_copy(
                src_ref=sub(xfull, cb, q), dst_ref=sub(xfull, cb, q),
                send_sem=agb_s.at[h, q], recv_sem=agb_r.at[h, q],
                device_id=(left,), device_id_type=pl.DeviceIdType.MESH,
            )

        xfull[pl.ds(i * M_PER, M_PER), :] = x_ref[...]
        agf_d, agb_d = {}, {}
        for q in range(SUB):
            agf_d[(0, q)] = mk_agf(0, q)
            agf_d[(0, q)].start()
            agb_d[(0, q)] = mk_agb(0, q)
            agb_d[(0, q)].start()
        compute(i)
        for h in range(1, F):
            for q in range(SUB):
                agf_d[(h - 1, q)].wait()
                agf_d[(h, q)] = mk_agf(h, q)
                agf_d[(h, q)].start()
                agb_d[(h - 1, q)].wait()
                if h < F - 1:
                    agb_d[(h, q)] = mk_agb(h, q)
                    agb_d[(h, q)].start()
            compute(perm_ref[(k - h) % N])
            compute(perm_ref[(k + h) % N])
        for q in range(SUB):
            agf_d[(F - 1, q)].wait()
        compute(perm_ref[(k - F) % N])

        def mk_rsf(h, q):
            return pltpu.make_async_remote_copy(
                src_ref=sub(facc, h, q), dst_ref=sub(facc, h + 1, q),
                send_sem=rsf_s.at[h, q], recv_sem=rsf_r.at[h, q],
                device_id=(right,), device_id_type=pl.DeviceIdType.MESH,
            )

        def mk_rsb(h, q):
            return pltpu.make_async_remote_copy(
                src_ref=sub(bacc, h, q), dst_ref=sub(bacc, h + 1, q),
                send_sem=rsb_s.at[h, q], recv_sem=rsb_r.at[h, q],
                device_id=(left,), device_id_type=pl.DeviceIdType.MESH,
            )

        facc[pl.ds(0, M_PER), :] = pbuf[
            pl.ds(perm_ref[(k + F - 1) % N] * M_PER, M_PER), :
        ]
        bacc[pl.ds(0, M_PER), :] = pbuf[
            pl.ds(perm_ref[(k - F) % N] * M_PER, M_PER), :
        ]
        rsf_d, rsb_d = {}, {}
        for q in range(SUB):
            rsf_d[(0, q)] = mk_rsf(0, q)
            rsf_d[(0, q)].start()
            rsb_d[(0, q)] = mk_rsb(0, q)
            rsb_d[(0, q)].start()
        for h in range(1, F):
            cf_r = perm_ref[(k + F - 1 - h) % N]
            cb_r = perm_ref[(k - F + h) % N]
            for q in range(SUB):
                rsf_d[(h - 1, q)].wait()
                if h < F - 1:
                    facc[pl.ds(h * M_PER + q * MS, MS), :] = (
                        facc[pl.ds(h * M_PER + q * MS, MS), :]
                        + pbuf[pl.ds(cf_r * M_PER + q * MS, MS), :]
                    )
                    rsf_d[(h, q)] = mk_rsf(h, q)
                    rsf_d[(h, q)].start()
                rsb_d[(h - 1, q)].wait()
                bacc[pl.ds(h * M_PER + q * MS, MS), :] = (
                    bacc[pl.ds(h * M_PER + q * MS, MS), :]
                    + pbuf[pl.ds(cb_r * M_PER + q * MS, MS), :]
                )
                rsb_d[(h, q)] = mk_rsb(h, q)
                rsb_d[(h, q)].start()
        for q in range(SUB):
            rsb_d[(F - 1, q)].wait()
        out_ref[...] = (
            facc[pl.ds((F - 1) * M_PER, M_PER), :].astype(jnp.float32)
            + bacc[pl.ds(F * M_PER, M_PER), :].astype(jnp.float32)
            + pbuf[pl.ds(i * M_PER, M_PER), :].astype(jnp.float32)
        )

    return pl.pallas_call(
        body,
        out_shape=jax.ShapeDtypeStruct((M_PER, D), jnp.float32),
        in_specs=[
            pl.BlockSpec(memory_space=pltpu.SMEM),
            pl.BlockSpec(memory_space=pltpu.SMEM),
            pl.BlockSpec(memory_space=pltpu.VMEM),
            pl.BlockSpec(memory_space=pltpu.VMEM),
            pl.BlockSpec(memory_space=pltpu.VMEM),
        ],
        out_specs=pl.BlockSpec(memory_space=pltpu.VMEM),
        scratch_shapes=[
            pltpu.VMEM((N * M_PER, D), jnp.bfloat16),
            pltpu.VMEM((N * M_PER, D), jnp.bfloat16),
            pltpu.VMEM((F * M_PER, D), jnp.bfloat16),
            pltpu.VMEM(((F + 1) * M_PER, D), jnp.bfloat16),
            pltpu.SemaphoreType.DMA((F, SUB)),
            pltpu.SemaphoreType.DMA((F, SUB)),
            pltpu.SemaphoreType.DMA((F - 1, SUB)),
            pltpu.SemaphoreType.DMA((F - 1, SUB)),
            pltpu.SemaphoreType.DMA((F - 1, SUB)),
            pltpu.SemaphoreType.DMA((F - 1, SUB)),
            pltpu.SemaphoreType.DMA((F, SUB)),
            pltpu.SemaphoreType.DMA((F, SUB)),
        ],
        compiler_params=pltpu.CompilerParams(
            collective_id=0,
            vmem_limit_bytes=60 * 1024 * 1024,
        ),
    )(kpos, perm, x16, w1, w2)


def kernel(x, W1, W2):
    i = lax.axis_index("i")
    kpos = jnp.take(jnp.array(INV, dtype=jnp.int32), i).reshape(1)
    perm = jnp.array(PERM, dtype=jnp.int32)
    return _fused(
        kpos,
        perm,
        x.astype(jnp.bfloat16),
        W1.astype(jnp.bfloat16),
        W2.astype(jnp.bfloat16),
    )
